# baseline (speedup 1.0000x reference)
import functools

import jax
import jax.numpy as jnp
from jax import lax
from jax.experimental import pallas as pl
from jax.experimental.pallas import tpu as pltpu
from jax.experimental.pallas import tpu_sc as plsc

_D = _H = _W = 256
_N = 1048576
_DHW = _D * _H * _W

_NC = 2
_NS = 16
_NW = _NC * _NS
_PER_W = _N // _NW
_CHUNK = 1024
_NCHUNK = _PER_W // _CHUNK

_OFFS = (0, 1, _W, _W + 1, _H * _W, _H * _W + 1, _H * _W + _W, _H * _W + _W + 1)


def _tec_body(cols_hbm, vol_hbm, out_hbm, cols_v, idx_v, w_v, val_v, out_v, sems):
    wid = lax.axis_index("s") * _NC + lax.axis_index("c")
    base_w = wid * _PER_W

    def prep(c, b):
        base = base_w + c * _CHUNK
        pltpu.sync_copy(cols_hbm.at[:, pl.ds(base, _CHUNK)], cols_v[b])

        def compute_idx(j, carry):
            dsl = pl.ds(j * 16, 16)
            f = cols_v[b][6, dsl]
            yin = f == 0.0
            x = jnp.where(yin, cols_v[b][0, dsl], cols_v[b][3, dsl])
            y = jnp.where(yin, cols_v[b][1, dsl], cols_v[b][4, dsl])
            z = jnp.where(yin, cols_v[b][2, dsl], cols_v[b][5, dsl])
            xf = (x + 1.0) * 0.5 * 255.0
            yf = (y + 1.0) * 0.5 * 255.0
            zf = (z + 1.0) * 0.5 * 255.0
            xi = jnp.minimum(jnp.maximum(xf.astype(jnp.int32), 0), 254)
            yi = jnp.minimum(jnp.maximum(yf.astype(jnp.int32), 0), 254)
            zi = jnp.minimum(jnp.maximum(zf.astype(jnp.int32), 0), 254)
            wx1 = xf - xi.astype(jnp.float32)
            wy1 = yf - yi.astype(jnp.float32)
            wz1 = zf - zi.astype(jnp.float32)
            wx0 = 1.0 - wx1
            wy0 = 1.0 - wy1
            wz0 = 1.0 - wz1
            fi = f.astype(jnp.int32) * _DHW
            i000 = fi + zi * (_H * _W) + yi * _W + xi
            for k in range(8):
                idx_v[b * 8 + k][dsl] = i000 + _OFFS[k]
            a = wy0 * wz0
            bb = wy1 * wz0
            cc = wy0 * wz1
            d = wy1 * wz1
            w_v[b][0, dsl] = wx0 * a
            w_v[b][1, dsl] = wx1 * a
            w_v[b][2, dsl] = wx0 * bb
            w_v[b][3, dsl] = wx1 * bb
            w_v[b][4, dsl] = wx0 * cc
            w_v[b][5, dsl] = wx1 * cc
            w_v[b][6, dsl] = wx0 * d
            w_v[b][7, dsl] = wx1 * d
            return carry

        lax.fori_loop(0, _CHUNK // 16, compute_idx, 0)
        for k in range(8):
            pltpu.async_copy(vol_hbm.at[idx_v[b * 8 + k]], val_v[b * 8 + k],
                             sems[b])

    def finish(c, b):
        for k in range(8):
            pltpu.make_async_copy(vol_hbm.at[idx_v[b * 8 + k]],
                                  val_v[b * 8 + k], sems[b]).wait()

        def compute_out(j, carry):
            dsl = pl.ds(j * 16, 16)
            acc = val_v[b * 8][dsl] * w_v[b][0, dsl]
            for k in range(1, 8):
                acc = acc + val_v[b * 8 + k][dsl] * w_v[b][k, dsl]
            out_v[b][dsl] = acc
            return carry

        lax.fori_loop(0, _CHUNK // 16, compute_out, 0)
        base = base_w + c * _CHUNK
        pltpu.sync_copy(out_v[b], out_hbm.at[pl.ds(base, _CHUNK)])

    prep(0, 0)

    def loop_body(i, carry):
        c = 2 * i
        prep(c + 1, 1)
        finish(c, 0)

        @pl.when(i < _NCHUNK // 2 - 1)
        def _():
            prep(c + 2, 0)

        finish(c + 1, 1)
        return carry

    lax.fori_loop(0, _NCHUNK // 2, loop_body, 0)


_sc_call = functools.partial(
    pl.kernel,
    out_type=jax.ShapeDtypeStruct((_N,), jnp.float32),
    mesh=plsc.VectorSubcoreMesh(core_axis_name="c", subcore_axis_name="s"),
    scratch_types=[
        [pltpu.VMEM((7, _CHUNK), jnp.float32) for _ in range(2)],
        [pltpu.VMEM((_CHUNK,), jnp.int32) for _ in range(16)],
        [pltpu.VMEM((8, _CHUNK), jnp.float32) for _ in range(2)],
        [pltpu.VMEM((_CHUNK,), jnp.float32) for _ in range(16)],
        [pltpu.VMEM((_CHUNK,), jnp.float32) for _ in range(2)],
        [pltpu.SemaphoreType.DMA for _ in range(2)],
    ],
)(_tec_body)


@jax.jit
def kernel(norm_samples, alpha_volume_yin, alpha_volume_yang):
    cols = norm_samples.T
    vol = jnp.concatenate([alpha_volume_yin, alpha_volume_yang], axis=0)
    vol = vol.reshape(-1)
    return _sc_call(cols, vol)


# speedup vs baseline: 1.0098x; 1.0098x over previous
import functools

import jax
import jax.numpy as jnp
from jax import lax
from jax.experimental import pallas as pl
from jax.experimental.pallas import tpu as pltpu
from jax.experimental.pallas import tpu_sc as plsc

_D = _H = _W = 256
_N = 1048576
_DHW = _D * _H * _W

_NC = 2
_NS = 16
_NW = _NC * _NS
_PER_W = _N // _NW
_CHUNK = 2048
_NCHUNK = _PER_W // _CHUNK

_OFFS = (0, 1, _W, _W + 1, _H * _W, _H * _W + 1, _H * _W + _W, _H * _W + _W + 1)


def _tec_body(rows_hbm, vol_hbm, out_hbm, rows_v, idx_v, w_v, val_v, out_v, sems):
    wid = lax.axis_index("s") * _NC + lax.axis_index("c")
    base_w = wid * _PER_W

    def prep(c, b):
        base = base_w + c * _CHUNK
        pltpu.sync_copy(rows_hbm.at[:, pl.ds(base, _CHUNK)], rows_v[b])

        def compute_idx(j, carry):
            dsl = pl.ds(j * 16, 16)
            f = rows_v[b][6, dsl]
            yin = f == 0.0
            x = jnp.where(yin, rows_v[b][0, dsl], rows_v[b][3, dsl])
            y = jnp.where(yin, rows_v[b][1, dsl], rows_v[b][4, dsl])
            z = jnp.where(yin, rows_v[b][2, dsl], rows_v[b][5, dsl])
            xf = (x + 1.0) * 0.5 * 255.0
            yf = (y + 1.0) * 0.5 * 255.0
            zf = (z + 1.0) * 0.5 * 255.0
            xi = jnp.minimum(jnp.maximum(xf.astype(jnp.int32), 0), 254)
            yi = jnp.minimum(jnp.maximum(yf.astype(jnp.int32), 0), 254)
            zi = jnp.minimum(jnp.maximum(zf.astype(jnp.int32), 0), 254)
            fi = f.astype(jnp.int32) * _DHW
            i000 = fi + zi * (_H * _W) + yi * _W + xi
            for k in range(8):
                idx_v[b * 8 + k][dsl] = i000 + _OFFS[k]
            w_v[b][0, dsl] = xf - xi.astype(jnp.float32)
            w_v[b][1, dsl] = yf - yi.astype(jnp.float32)
            w_v[b][2, dsl] = zf - zi.astype(jnp.float32)
            return carry

        lax.fori_loop(0, _CHUNK // 16, compute_idx, 0)
        for k in range(8):
            pltpu.async_copy(vol_hbm.at[idx_v[b * 8 + k]], val_v[b * 8 + k],
                             sems[b])

    def finish(c, b):
        for k in range(8):
            pltpu.make_async_copy(vol_hbm.at[idx_v[b * 8 + k]],
                                  val_v[b * 8 + k], sems[b]).wait()

        def compute_out(j, carry):
            dsl = pl.ds(j * 16, 16)
            wx1 = w_v[b][0, dsl]
            wy1 = w_v[b][1, dsl]
            wz1 = w_v[b][2, dsl]
            wx0 = 1.0 - wx1
            wy0 = 1.0 - wy1
            wz0 = 1.0 - wz1
            a = wy0 * wz0
            bb = wy1 * wz0
            cc = wy0 * wz1
            d = wy1 * wz1
            acc = val_v[b * 8 + 0][dsl] * (wx0 * a)
            acc = acc + val_v[b * 8 + 1][dsl] * (wx1 * a)
            acc = acc + val_v[b * 8 + 2][dsl] * (wx0 * bb)
            acc = acc + val_v[b * 8 + 3][dsl] * (wx1 * bb)
            acc = acc + val_v[b * 8 + 4][dsl] * (wx0 * cc)
            acc = acc + val_v[b * 8 + 5][dsl] * (wx1 * cc)
            acc = acc + val_v[b * 8 + 6][dsl] * (wx0 * d)
            acc = acc + val_v[b * 8 + 7][dsl] * (wx1 * d)
            out_v[b][dsl] = acc
            return carry

        lax.fori_loop(0, _CHUNK // 16, compute_out, 0)
        base = base_w + c * _CHUNK
        pltpu.sync_copy(out_v[b], out_hbm.at[pl.ds(base, _CHUNK)])

    prep(0, 0)

    def loop_body(i, carry):
        c = 2 * i
        prep(c + 1, 1)
        finish(c, 0)

        @pl.when(i < _NCHUNK // 2 - 1)
        def _():
            prep(c + 2, 0)

        finish(c + 1, 1)
        return carry

    lax.fori_loop(0, _NCHUNK // 2, loop_body, 0)


_sc_call = functools.partial(
    pl.kernel,
    out_type=jax.ShapeDtypeStruct((_N,), jnp.float32),
    mesh=plsc.VectorSubcoreMesh(core_axis_name="c", subcore_axis_name="s"),
    scratch_types=[
        [pltpu.VMEM((7, _CHUNK), jnp.float32) for _ in range(2)],
        [pltpu.VMEM((_CHUNK,), jnp.int32) for _ in range(16)],
        [pltpu.VMEM((3, _CHUNK), jnp.float32) for _ in range(2)],
        [pltpu.VMEM((_CHUNK,), jnp.float32) for _ in range(16)],
        [pltpu.VMEM((_CHUNK,), jnp.float32) for _ in range(2)],
        [pltpu.SemaphoreType.DMA for _ in range(2)],
    ],
)(_tec_body)


@jax.jit
def kernel(norm_samples, alpha_volume_yin, alpha_volume_yang):
    rows = norm_samples.T
    vol = jnp.concatenate([alpha_volume_yin, alpha_volume_yang], axis=0)
    vol = vol.reshape(-1)
    return _sc_call(rows, vol)
